# Initial kernel scaffold; baseline (speedup 1.0000x reference)
import jax
import jax.numpy as jnp
from jax.experimental import pallas as pl


def kernel(adj_t, x):
    raise NotImplementedError("write your pallas kernel here")


# speedup vs baseline: 8.4771x; 8.4771x over previous
import functools

import jax
import jax.numpy as jnp
from jax import lax
from jax.experimental import pallas as pl
from jax.experimental.pallas import tpu as pltpu
from jax.experimental.pallas import tpu_sc as plsc

N = 10000
E = 320000
D = 128

NC = 2
NS = 16
NW = NC * NS

CHUNK = 128
N_PAD = 10240
RPT = N_PAD // NS
EPW = 10240
NCHUNK = EPW // CHUNK
NVEC = EPW // 16
E_PAD = NW * EPW

_MESH = plsc.VectorSubcoreMesh(core_axis_name="c", subcore_axis_name="s")


@functools.partial(
    pl.kernel,
    out_type=jax.ShapeDtypeStruct((NW, 2, N_PAD), jnp.float32),
    mesh=_MESH,
    compiler_params=pltpu.CompilerParams(needs_layout_passes=False),
    scratch_types=[
        pltpu.VMEM((EPW,), jnp.int32),
        pltpu.VMEM((EPW,), jnp.int32),
        pltpu.VMEM((N_PAD,), jnp.float32),
        pltpu.VMEM((N_PAD,), jnp.float32),
    ],
)
def _deg_kernel(src_hbm, dst_hbm, zeros_hbm, hp_hbm,
                src_v, dst_v, hist_s, hist_d):
    cid = lax.axis_index("c")
    sid = lax.axis_index("s")
    wid = cid * NS + sid

    pltpu.sync_copy(src_hbm.at[wid], src_v)
    pltpu.sync_copy(dst_hbm.at[wid], dst_v)
    pltpu.sync_copy(zeros_hbm, hist_s)
    pltpu.sync_copy(zeros_hbm, hist_d)
    ones = jnp.full((16,), 1.0, jnp.float32)

    def body(j, carry):
        plsc.addupdate_scatter(hist_s, [src_v[pl.ds(j * 16, 16)]], ones)
        plsc.addupdate_scatter(hist_d, [dst_v[pl.ds(j * 16, 16)]], ones)
        return carry

    lax.fori_loop(0, NVEC, body, 0)
    pltpu.sync_copy(hist_s, hp_hbm.at[wid, 0])
    pltpu.sync_copy(hist_d, hp_hbm.at[wid, 1])


@functools.partial(
    pl.kernel,
    out_type=jax.ShapeDtypeStruct((NC, N_PAD, D), jnp.float32),
    mesh=_MESH,
    scratch_types=[
        pltpu.VMEM((NCHUNK, CHUNK), jnp.int32),
        pltpu.VMEM((NCHUNK, CHUNK), jnp.int32),
        pltpu.VMEM((CHUNK, D), jnp.float32),
        pltpu.VMEM_SHARED((N_PAD, D), jnp.float32),
        pltpu.SemaphoreType.DMA,
    ],
)
def _edge_kernel(src_hbm, dst_hbm, x_hbm, zeros_hbm, p_hbm,
                 src_v, dst_v, rows_v, acc, gsem):
    cid = lax.axis_index("c")
    sid = lax.axis_index("s")
    wid = cid * NS + sid

    pltpu.sync_copy(src_hbm.at[wid], src_v)
    pltpu.sync_copy(dst_hbm.at[wid], dst_v)
    rows = pl.ds(sid * RPT, RPT)
    pltpu.sync_copy(zeros_hbm.at[rows], acc.at[rows])
    plsc.subcore_barrier()

    def body(j, carry):
        pltpu.async_copy(x_hbm.at[src_v.at[j]], rows_v, gsem).wait()
        pltpu.sync_copy(rows_v, acc.at[dst_v.at[j]], add=True)
        return carry

    lax.fori_loop(0, NCHUNK, body, 0)
    plsc.subcore_barrier()

    pltpu.sync_copy(acc.at[rows], p_hbm.at[cid, rows])


def _norm_body(hp_ref, no_ref, smid_ref, sfin_ref):
    h = jnp.sum(hp_ref[...], axis=0)
    no = lax.rsqrt(jnp.clip(h[0], 1.0, None))
    ni = lax.rsqrt(jnp.clip(h[1], 1.0, None))
    no_ref[...] = no
    smid_ref[...] = no * ni
    sfin_ref[...] = ni


def _norms(hp):
    shp = jax.ShapeDtypeStruct((N_PAD // 128, 128), jnp.float32)
    return pl.pallas_call(
        _norm_body,
        out_shape=[shp, shp, shp],
    )(hp)


def _prescale_body(x_ref, s_ref, out_ref):
    out_ref[...] = x_ref[...] * s_ref[...]


def _prescale(x_pad, s_col):
    return pl.pallas_call(
        _prescale_body,
        grid=(NS,),
        in_specs=[
            pl.BlockSpec((RPT, D), lambda i: (i, 0)),
            pl.BlockSpec((RPT, 1), lambda i: (i, 0)),
        ],
        out_specs=pl.BlockSpec((RPT, D), lambda i: (i, 0)),
        out_shape=jax.ShapeDtypeStruct((N_PAD, D), jnp.float32),
    )(x_pad, s_col)


def _combine_body(p_ref, s_ref, out_ref):
    pb = p_ref[...]
    out_ref[...] = (pb[0] + pb[1]) * s_ref[...]


def _combine(p, s_col):
    return pl.pallas_call(
        _combine_body,
        grid=(NS,),
        in_specs=[
            pl.BlockSpec((NC, RPT, D), lambda i: (0, i, 0)),
            pl.BlockSpec((RPT, 1), lambda i: (i, 0)),
        ],
        out_specs=pl.BlockSpec((RPT, D), lambda i: (i, 0)),
        out_shape=jax.ShapeDtypeStruct((N_PAD, D), jnp.float32),
    )(p, s_col)


def kernel(adj_t, x):
    src = adj_t[0].astype(jnp.int32)
    dst = adj_t[1].astype(jnp.int32)
    pad_ids = N + (jnp.arange(E_PAD - E, dtype=jnp.int32) % 128)
    src_flat = jnp.concatenate([src, pad_ids]).reshape(NW, EPW)
    dst_flat = jnp.concatenate([dst, pad_ids]).reshape(NW, EPW)
    src_p = src_flat.reshape(NW, NCHUNK, CHUNK)
    dst_p = dst_flat.reshape(NW, NCHUNK, CHUNK)
    x_pad = jnp.concatenate(
        [x.astype(jnp.float32), jnp.zeros((N_PAD - N, D), jnp.float32)])

    zeros_n = jnp.zeros((N_PAD,), jnp.float32)
    zeros_nd = jnp.zeros((N_PAD, D), jnp.float32)

    hp = _deg_kernel(src_flat, dst_flat, zeros_n)
    no, smid, sfin = _norms(hp.reshape(NW, 2, N_PAD // 128, 128))
    no_col = no.reshape(N_PAD, 1)
    smid_col = smid.reshape(N_PAD, 1)
    sfin_col = sfin.reshape(N_PAD, 1)

    x1s = _prescale(x_pad, no_col)
    p1 = _edge_kernel(src_p, dst_p, x1s, zeros_nd)
    x2s = _combine(p1, smid_col)
    p2 = _edge_kernel(src_p, dst_p, x2s, zeros_nd)
    out_pad = _combine(p2, sfin_col)
    return out_pad[:N]


# speedup vs baseline: 11.6979x; 1.3799x over previous
import functools

import jax
import jax.numpy as jnp
from jax import lax
from jax.experimental import pallas as pl
from jax.experimental.pallas import tpu as pltpu
from jax.experimental.pallas import tpu_sc as plsc

N = 10000
E = 320000
D = 128

NC = 2
NS = 16
NW = NC * NS

CHUNK = 128
N_PAD = 10240
RPT = N_PAD // NS
EPW = 10240
NCHUNK = EPW // CHUNK
IG = 8
NG = NCHUNK // IG
NVEC = EPW // 16
E_PAD = NW * EPW

_MESH = plsc.VectorSubcoreMesh(core_axis_name="c", subcore_axis_name="s")


@functools.partial(
    pl.kernel,
    out_type=jax.ShapeDtypeStruct((NW, 2, N_PAD), jnp.float32),
    mesh=_MESH,
    compiler_params=pltpu.CompilerParams(needs_layout_passes=False),
    scratch_types=[
        pltpu.VMEM((EPW,), jnp.int32),
        pltpu.VMEM((EPW,), jnp.int32),
        pltpu.VMEM((N_PAD,), jnp.float32),
        pltpu.VMEM((N_PAD,), jnp.float32),
    ],
)
def _deg_kernel(src_hbm, dst_hbm, zeros_hbm, hp_hbm,
                src_v, dst_v, hist_s, hist_d):
    cid = lax.axis_index("c")
    sid = lax.axis_index("s")
    wid = cid * NS + sid

    pltpu.sync_copy(src_hbm.at[wid], src_v)
    pltpu.sync_copy(dst_hbm.at[wid], dst_v)
    pltpu.sync_copy(zeros_hbm, hist_s)
    pltpu.sync_copy(zeros_hbm, hist_d)
    ones = jnp.full((16,), 1.0, jnp.float32)

    def body(j, carry):
        plsc.addupdate_scatter(hist_s, [src_v[pl.ds(j * 16, 16)]], ones)
        plsc.addupdate_scatter(hist_d, [dst_v[pl.ds(j * 16, 16)]], ones)
        return carry

    lax.fori_loop(0, NVEC, body, 0)
    pltpu.sync_copy(hist_s, hp_hbm.at[wid, 0])
    pltpu.sync_copy(hist_d, hp_hbm.at[wid, 1])


NBUF = 2

@functools.partial(
    pl.kernel,
    out_type=jax.ShapeDtypeStruct((NC, N_PAD, D), jnp.float32),
    mesh=_MESH,
    scratch_types=[
        pltpu.VMEM((2, IG, CHUNK), jnp.int32),
        pltpu.VMEM((2, IG, CHUNK), jnp.int32),
        pltpu.VMEM((CHUNK, D), jnp.float32),
        pltpu.VMEM((CHUNK, D), jnp.float32),
        pltpu.VMEM_SHARED((N_PAD, D), jnp.float32),
        pltpu.SemaphoreType.DMA,
        pltpu.SemaphoreType.DMA,
    ],
)
def _edge_kernel(ids_hbm, x_hbm, zeros_hbm, p_hbm,
                 idb_0, idb_1, rows_0, rows_1, acc, gsem_0, gsem_1):
    idb = (idb_0, idb_1)
    rows_v = (rows_0, rows_1)
    gsem = (gsem_0, gsem_1)
    cid = lax.axis_index("c")
    sid = lax.axis_index("s")
    wid = cid * NS + sid

    rows = pl.ds(sid * RPT, RPT)
    pltpu.sync_copy(zeros_hbm.at[rows], acc.at[rows])
    pltpu.sync_copy(ids_hbm.at[wid, 0], idb_0)
    pltpu.sync_copy(ids_hbm.at[wid, 1], idb_1)
    plsc.subcore_barrier()

    for b in range(NBUF):
        pltpu.async_copy(x_hbm.at[idb_0.at[0, b]], rows_v[b], gsem[b])

    def body(p, carry):
        for gg in range(2):
            idc = idb[gg]
            idn = idb[1 - gg]
            for k in range(IG):
                b = k % 2
                pltpu.make_async_copy(
                    x_hbm.at[idc.at[0, k]], rows_v[b], gsem[b]).wait()
                pltpu.sync_copy(rows_v[b], acc.at[idc.at[1, k]], add=True)
                if k < IG - NBUF:
                    pltpu.async_copy(
                        x_hbm.at[idc.at[0, k + NBUF]], rows_v[b], gsem[b])
                else:
                    pltpu.async_copy(
                        x_hbm.at[idn.at[0, k + NBUF - IG]], rows_v[b],
                        gsem[b])
            gnext = jnp.minimum(2 * p + 2 + gg, NG - 1)
            pltpu.sync_copy(ids_hbm.at[wid, gnext], idc)
        return carry

    lax.fori_loop(0, NG // 2, body, 0)

    for b in range(NBUF):
        pltpu.make_async_copy(
            x_hbm.at[idb_0.at[0, b]], rows_v[b], gsem[b]).wait()

    plsc.subcore_barrier()
    pltpu.sync_copy(acc.at[rows], p_hbm.at[cid, rows])


def _norm_body(hp_ref, no_ref, smid_ref, sfin_ref):
    h = jnp.sum(hp_ref[...], axis=0)
    no = lax.rsqrt(jnp.clip(h[0], 1.0, None))
    ni = lax.rsqrt(jnp.clip(h[1], 1.0, None))
    no_ref[...] = no
    smid_ref[...] = no * ni
    sfin_ref[...] = ni


def _norms(hp):
    shp = jax.ShapeDtypeStruct((N_PAD // 128, 128), jnp.float32)
    return pl.pallas_call(
        _norm_body,
        out_shape=[shp, shp, shp],
    )(hp)


def _prescale_body(x_ref, s_ref, out_ref):
    out_ref[...] = x_ref[...] * s_ref[...]


def _prescale(x_pad, s_col):
    return pl.pallas_call(
        _prescale_body,
        grid=(NS,),
        in_specs=[
            pl.BlockSpec((RPT, D), lambda i: (i, 0)),
            pl.BlockSpec((RPT, 1), lambda i: (i, 0)),
        ],
        out_specs=pl.BlockSpec((RPT, D), lambda i: (i, 0)),
        out_shape=jax.ShapeDtypeStruct((N_PAD, D), jnp.float32),
    )(x_pad, s_col)


def _combine_body(p_ref, s_ref, out_ref):
    pb = p_ref[...]
    out_ref[...] = (pb[0] + pb[1]) * s_ref[...]


def _combine(p, s_col):
    return pl.pallas_call(
        _combine_body,
        grid=(NS,),
        in_specs=[
            pl.BlockSpec((NC, RPT, D), lambda i: (0, i, 0)),
            pl.BlockSpec((RPT, 1), lambda i: (i, 0)),
        ],
        out_specs=pl.BlockSpec((RPT, D), lambda i: (i, 0)),
        out_shape=jax.ShapeDtypeStruct((N_PAD, D), jnp.float32),
    )(p, s_col)


def kernel(adj_t, x):
    src = adj_t[0].astype(jnp.int32)
    dst = adj_t[1].astype(jnp.int32)
    pad_ids = N + (jnp.arange(E_PAD - E, dtype=jnp.int32) % 128)
    src_flat = jnp.concatenate([src, pad_ids]).reshape(NW, EPW)
    dst_flat = jnp.concatenate([dst, pad_ids]).reshape(NW, EPW)
    ids_p = jnp.stack([src_flat.reshape(NW, NG, IG, CHUNK),
                       dst_flat.reshape(NW, NG, IG, CHUNK)], axis=2)
    x_pad = jnp.concatenate(
        [x.astype(jnp.float32), jnp.zeros((N_PAD - N, D), jnp.float32)])

    zeros_n = jnp.zeros((N_PAD,), jnp.float32)
    zeros_nd = jnp.zeros((N_PAD, D), jnp.float32)

    hp = _deg_kernel(src_flat, dst_flat, zeros_n)
    no, smid, sfin = _norms(hp.reshape(NW, 2, N_PAD // 128, 128))
    no_col = no.reshape(N_PAD, 1)
    smid_col = smid.reshape(N_PAD, 1)
    sfin_col = sfin.reshape(N_PAD, 1)

    x1s = _prescale(x_pad, no_col)
    p1 = _edge_kernel(ids_p, x1s, zeros_nd)
    x2s = _combine(p1, smid_col)
    p2 = _edge_kernel(ids_p, x2s, zeros_nd)
    out_pad = _combine(p2, sfin_col)
    return out_pad[:N]


# speedup vs baseline: 12.2558x; 1.0477x over previous
import functools

import jax
import jax.numpy as jnp
from jax import lax
from jax.experimental import pallas as pl
from jax.experimental.pallas import tpu as pltpu
from jax.experimental.pallas import tpu_sc as plsc

N = 10000
E = 320000
D = 128

NC = 2
NS = 16
NW = NC * NS

CHUNK = 80
N_PAD = 10240
RPT = N_PAD // NS
EPW = 10240
NCHUNK = EPW // CHUNK
IG = 8
NG = NCHUNK // IG
NVEC = EPW // 16
E_PAD = NW * EPW

_MESH = plsc.VectorSubcoreMesh(core_axis_name="c", subcore_axis_name="s")


@functools.partial(
    pl.kernel,
    out_type=jax.ShapeDtypeStruct((NW, 2, N_PAD), jnp.float32),
    mesh=_MESH,
    compiler_params=pltpu.CompilerParams(needs_layout_passes=False),
    scratch_types=[
        pltpu.VMEM((EPW,), jnp.int32),
        pltpu.VMEM((EPW,), jnp.int32),
        pltpu.VMEM((N_PAD,), jnp.float32),
        pltpu.VMEM((N_PAD,), jnp.float32),
    ],
)
def _deg_kernel(src_hbm, dst_hbm, zeros_hbm, hp_hbm,
                src_v, dst_v, hist_s, hist_d):
    cid = lax.axis_index("c")
    sid = lax.axis_index("s")
    wid = cid * NS + sid

    pltpu.sync_copy(src_hbm.at[wid], src_v)
    pltpu.sync_copy(dst_hbm.at[wid], dst_v)
    pltpu.sync_copy(zeros_hbm, hist_s)
    pltpu.sync_copy(zeros_hbm, hist_d)
    ones = jnp.full((16,), 1.0, jnp.float32)

    def body(j, carry):
        plsc.addupdate_scatter(hist_s, [src_v[pl.ds(j * 16, 16)]], ones)
        plsc.addupdate_scatter(hist_d, [dst_v[pl.ds(j * 16, 16)]], ones)
        return carry

    lax.fori_loop(0, NVEC, body, 0)
    pltpu.sync_copy(hist_s, hp_hbm.at[wid, 0])
    pltpu.sync_copy(hist_d, hp_hbm.at[wid, 1])


NBUF = 4

@functools.partial(
    pl.kernel,
    out_type=jax.ShapeDtypeStruct((NC, N_PAD, D), jnp.float32),
    mesh=_MESH,
    scratch_types=[
        pltpu.VMEM((2, IG, CHUNK), jnp.int32),
        pltpu.VMEM((2, IG, CHUNK), jnp.int32),
        pltpu.VMEM((CHUNK, D), jnp.float32),
        pltpu.VMEM((CHUNK, D), jnp.float32),
        pltpu.VMEM((CHUNK, D), jnp.float32),
        pltpu.VMEM((CHUNK, D), jnp.float32),
        pltpu.VMEM_SHARED((N_PAD, D), jnp.float32),
        pltpu.SemaphoreType.DMA,
        pltpu.SemaphoreType.DMA,
        pltpu.SemaphoreType.DMA,
        pltpu.SemaphoreType.DMA,
    ],
)
def _edge_kernel(ids_hbm, x_hbm, zeros_hbm, p_hbm,
                 idb_0, idb_1, rows_0, rows_1, rows_2, rows_3, acc,
                 gsem_0, gsem_1, gsem_2, gsem_3):
    idb = (idb_0, idb_1)
    rows_v = (rows_0, rows_1, rows_2, rows_3)
    gsem = (gsem_0, gsem_1, gsem_2, gsem_3)
    cid = lax.axis_index("c")
    sid = lax.axis_index("s")
    wid = cid * NS + sid

    rows = pl.ds(sid * RPT, RPT)
    pltpu.sync_copy(zeros_hbm.at[rows], acc.at[rows])
    pltpu.sync_copy(ids_hbm.at[wid, 0], idb_0)
    pltpu.sync_copy(ids_hbm.at[wid, 1], idb_1)
    plsc.subcore_barrier()

    for b in range(NBUF):
        pltpu.async_copy(x_hbm.at[idb_0.at[0, b]], rows_v[b], gsem[b])

    def body(p, carry):
        for gg in range(2):
            idc = idb[gg]
            idn = idb[1 - gg]
            for k in range(IG):
                b = k % NBUF
                pltpu.make_async_copy(
                    x_hbm.at[idc.at[0, k]], rows_v[b], gsem[b]).wait()
                pltpu.sync_copy(rows_v[b], acc.at[idc.at[1, k]], add=True)
                if k < IG - NBUF:
                    pltpu.async_copy(
                        x_hbm.at[idc.at[0, k + NBUF]], rows_v[b], gsem[b])
                else:
                    pltpu.async_copy(
                        x_hbm.at[idn.at[0, k + NBUF - IG]], rows_v[b],
                        gsem[b])
            gnext = jnp.minimum(2 * p + 2 + gg, NG - 1)
            pltpu.sync_copy(ids_hbm.at[wid, gnext], idc)
        return carry

    lax.fori_loop(0, NG // 2, body, 0)

    for b in range(NBUF):
        pltpu.make_async_copy(
            x_hbm.at[idb_0.at[0, b]], rows_v[b], gsem[b]).wait()

    plsc.subcore_barrier()
    pltpu.sync_copy(acc.at[rows], p_hbm.at[cid, rows])


def _norm_body(hp_ref, no_ref, smid_ref, sfin_ref):
    h = jnp.sum(hp_ref[...], axis=0)
    no = lax.rsqrt(jnp.clip(h[0], 1.0, None))
    ni = lax.rsqrt(jnp.clip(h[1], 1.0, None))
    no_ref[...] = no
    smid_ref[...] = no * ni
    sfin_ref[...] = ni


def _norms(hp):
    shp = jax.ShapeDtypeStruct((N_PAD // 128, 128), jnp.float32)
    return pl.pallas_call(
        _norm_body,
        out_shape=[shp, shp, shp],
    )(hp)


def _prescale_body(x_ref, s_ref, out_ref):
    out_ref[...] = x_ref[...] * s_ref[...]


def _prescale(x_pad, s_col):
    return pl.pallas_call(
        _prescale_body,
        grid=(NS,),
        in_specs=[
            pl.BlockSpec((RPT, D), lambda i: (i, 0)),
            pl.BlockSpec((RPT, 1), lambda i: (i, 0)),
        ],
        out_specs=pl.BlockSpec((RPT, D), lambda i: (i, 0)),
        out_shape=jax.ShapeDtypeStruct((N_PAD, D), jnp.float32),
    )(x_pad, s_col)


def _combine_body(p_ref, s_ref, out_ref):
    pb = p_ref[...]
    out_ref[...] = (pb[0] + pb[1]) * s_ref[...]


def _combine(p, s_col):
    return pl.pallas_call(
        _combine_body,
        grid=(NS,),
        in_specs=[
            pl.BlockSpec((NC, RPT, D), lambda i: (0, i, 0)),
            pl.BlockSpec((RPT, 1), lambda i: (i, 0)),
        ],
        out_specs=pl.BlockSpec((RPT, D), lambda i: (i, 0)),
        out_shape=jax.ShapeDtypeStruct((N_PAD, D), jnp.float32),
    )(p, s_col)


def kernel(adj_t, x):
    src = adj_t[0].astype(jnp.int32)
    dst = adj_t[1].astype(jnp.int32)
    pad_ids = N + (jnp.arange(E_PAD - E, dtype=jnp.int32) % 128)
    src_flat = jnp.concatenate([src, pad_ids]).reshape(NW, EPW)
    dst_flat = jnp.concatenate([dst, pad_ids]).reshape(NW, EPW)
    ids_p = jnp.stack([src_flat.reshape(NW, NG, IG, CHUNK),
                       dst_flat.reshape(NW, NG, IG, CHUNK)], axis=2)
    x_pad = jnp.concatenate(
        [x.astype(jnp.float32), jnp.zeros((N_PAD - N, D), jnp.float32)])

    zeros_n = jnp.zeros((N_PAD,), jnp.float32)
    zeros_nd = jnp.zeros((N_PAD, D), jnp.float32)

    hp = _deg_kernel(src_flat, dst_flat, zeros_n)
    no, smid, sfin = _norms(hp.reshape(NW, 2, N_PAD // 128, 128))
    no_col = no.reshape(N_PAD, 1)
    smid_col = smid.reshape(N_PAD, 1)
    sfin_col = sfin.reshape(N_PAD, 1)

    x1s = _prescale(x_pad, no_col)
    p1 = _edge_kernel(ids_p, x1s, zeros_nd)
    x2s = _combine(p1, smid_col)
    p2 = _edge_kernel(ids_p, x2s, zeros_nd)
    out_pad = _combine(p2, sfin_col)
    return out_pad[:N]


# speedup vs baseline: 12.6001x; 1.0281x over previous
import functools

import jax
import jax.numpy as jnp
from jax import lax
from jax.experimental import pallas as pl
from jax.experimental.pallas import tpu as pltpu
from jax.experimental.pallas import tpu_sc as plsc

N = 10000
E = 320000
D = 128

NC = 2
NS = 16
NW = NC * NS

CHUNK = 80
N_PAD = 10240
RPT = N_PAD // NS
EPW = 10240
NCHUNK = EPW // CHUNK
IG = 8
NG = NCHUNK // IG
NVEC = EPW // 16
E_PAD = NW * EPW

_MESH = plsc.VectorSubcoreMesh(core_axis_name="c", subcore_axis_name="s")


@functools.partial(
    pl.kernel,
    out_type=jax.ShapeDtypeStruct((NW, 2, N_PAD), jnp.float32),
    mesh=_MESH,
    compiler_params=pltpu.CompilerParams(needs_layout_passes=False),
    scratch_types=[
        pltpu.VMEM((EPW,), jnp.int32),
        pltpu.VMEM((EPW,), jnp.int32),
        pltpu.VMEM((N_PAD,), jnp.float32),
        pltpu.VMEM((N_PAD,), jnp.float32),
    ],
)
def _deg_kernel(src_hbm, dst_hbm, zeros_hbm, hp_hbm,
                src_v, dst_v, hist_s, hist_d):
    cid = lax.axis_index("c")
    sid = lax.axis_index("s")
    wid = cid * NS + sid

    pltpu.sync_copy(src_hbm.at[wid], src_v)
    pltpu.sync_copy(dst_hbm.at[wid], dst_v)
    pltpu.sync_copy(zeros_hbm, hist_s)
    pltpu.sync_copy(zeros_hbm, hist_d)
    ones = jnp.full((16,), 1.0, jnp.float32)

    def body(j, carry):
        plsc.addupdate_scatter(hist_s, [src_v[pl.ds(j * 16, 16)]], ones)
        plsc.addupdate_scatter(hist_d, [dst_v[pl.ds(j * 16, 16)]], ones)
        return carry

    lax.fori_loop(0, NVEC, body, 0)
    pltpu.sync_copy(hist_s, hp_hbm.at[wid, 0])
    pltpu.sync_copy(hist_d, hp_hbm.at[wid, 1])


NBUF = 4

@functools.partial(
    pl.kernel,
    out_type=jax.ShapeDtypeStruct((NC, N_PAD, D), jnp.float32),
    mesh=_MESH,
    scratch_types=[
        pltpu.VMEM((2, IG, CHUNK), jnp.int32),
        pltpu.VMEM((2, IG, CHUNK), jnp.int32),
        pltpu.VMEM((CHUNK, D), jnp.float32),
        pltpu.VMEM((CHUNK, D), jnp.float32),
        pltpu.VMEM((CHUNK, D), jnp.float32),
        pltpu.VMEM((CHUNK, D), jnp.float32),
        pltpu.VMEM_SHARED((N_PAD, D), jnp.float32),
        pltpu.SemaphoreType.DMA,
        pltpu.SemaphoreType.DMA,
        pltpu.SemaphoreType.DMA,
        pltpu.SemaphoreType.DMA,
    ],
)
def _edge_kernel(ids_hbm, x_hbm, zeros_hbm, p_hbm,
                 idb_0, idb_1, rows_0, rows_1, rows_2, rows_3, acc,
                 gsem_0, gsem_1, gsem_2, gsem_3):
    idb = (idb_0, idb_1)
    rows_v = (rows_0, rows_1, rows_2, rows_3)
    gsem = (gsem_0, gsem_1, gsem_2, gsem_3)
    cid = lax.axis_index("c")
    sid = lax.axis_index("s")
    wid = cid * NS + sid

    rows = pl.ds(sid * RPT, RPT)
    pltpu.sync_copy(zeros_hbm.at[rows], acc.at[rows])
    pltpu.sync_copy(ids_hbm.at[wid, 0], idb_0)
    pltpu.sync_copy(ids_hbm.at[wid, 1], idb_1)
    plsc.subcore_barrier()

    for b in range(NBUF):
        pltpu.async_copy(x_hbm.at[idb_0.at[0, b]], rows_v[b], gsem[b])

    def body(p, carry):
        for gg in range(2):
            idc = idb[gg]
            idn = idb[1 - gg]
            for k in range(IG):
                b = k % NBUF
                pltpu.make_async_copy(
                    x_hbm.at[idc.at[0, k]], rows_v[b], gsem[b]).wait()
                pltpu.sync_copy(rows_v[b], acc.at[idc.at[1, k]], add=True)
                if k < IG - NBUF:
                    pltpu.async_copy(
                        x_hbm.at[idc.at[0, k + NBUF]], rows_v[b], gsem[b])
                else:
                    pltpu.async_copy(
                        x_hbm.at[idn.at[0, k + NBUF - IG]], rows_v[b],
                        gsem[b])
            gnext = jnp.minimum(2 * p + 2 + gg, NG - 1)
            pltpu.sync_copy(ids_hbm.at[wid, gnext], idc)
        return carry

    lax.fori_loop(0, NG // 2, body, 0)

    for b in range(NBUF):
        pltpu.make_async_copy(
            x_hbm.at[idb_0.at[0, b]], rows_v[b], gsem[b]).wait()

    plsc.subcore_barrier()
    pltpu.sync_copy(acc.at[rows], p_hbm.at[cid, rows])


def _norm_prescale_body(hp_ref, x_ref, x1s_ref, smid_ref, sfin_ref):
    h = jnp.sum(hp_ref[...], axis=0)
    no = lax.rsqrt(jnp.clip(h[0], 1.0, None))
    ni = lax.rsqrt(jnp.clip(h[1], 1.0, None))
    x1s_ref[...] = x_ref[...] * no[:, None]
    smid_ref[...] = (no * ni)[:, None]
    sfin_ref[...] = ni[:, None]


def _norm_prescale(hp, x_pad):
    col = jax.ShapeDtypeStruct((N_PAD, 1), jnp.float32)
    return pl.pallas_call(
        _norm_prescale_body,
        grid=(NS,),
        in_specs=[
            pl.BlockSpec((NW, 2, RPT), lambda i: (0, 0, i)),
            pl.BlockSpec((RPT, D), lambda i: (i, 0)),
        ],
        out_specs=[
            pl.BlockSpec((RPT, D), lambda i: (i, 0)),
            pl.BlockSpec((RPT, 1), lambda i: (i, 0)),
            pl.BlockSpec((RPT, 1), lambda i: (i, 0)),
        ],
        out_shape=[jax.ShapeDtypeStruct((N_PAD, D), jnp.float32), col, col],
    )(hp, x_pad)


def _combine_body(p_ref, s_ref, out_ref):
    pb = p_ref[...]
    out_ref[...] = (pb[0] + pb[1]) * s_ref[...]


def _combine(p, s_col):
    return pl.pallas_call(
        _combine_body,
        grid=(NS,),
        in_specs=[
            pl.BlockSpec((NC, RPT, D), lambda i: (0, i, 0)),
            pl.BlockSpec((RPT, 1), lambda i: (i, 0)),
        ],
        out_specs=pl.BlockSpec((RPT, D), lambda i: (i, 0)),
        out_shape=jax.ShapeDtypeStruct((N_PAD, D), jnp.float32),
    )(p, s_col)


def kernel(adj_t, x):
    src = adj_t[0].astype(jnp.int32)
    dst = adj_t[1].astype(jnp.int32)
    pad_ids = N + (jnp.arange(E_PAD - E, dtype=jnp.int32) % 128)
    src_flat = jnp.concatenate([src, pad_ids]).reshape(NW, EPW)
    dst_flat = jnp.concatenate([dst, pad_ids]).reshape(NW, EPW)
    ids_p = jnp.stack([src_flat.reshape(NW, NG, IG, CHUNK),
                       dst_flat.reshape(NW, NG, IG, CHUNK)], axis=2)
    x_pad = jnp.concatenate(
        [x.astype(jnp.float32), jnp.zeros((N_PAD - N, D), jnp.float32)])

    zeros_n = jnp.zeros((N_PAD,), jnp.float32)
    zeros_nd = jnp.zeros((N_PAD, D), jnp.float32)

    hp = _deg_kernel(src_flat, dst_flat, zeros_n)
    x1s, smid_col, sfin_col = _norm_prescale(hp, x_pad)
    p1 = _edge_kernel(ids_p, x1s, zeros_nd)
    x2s = _combine(p1, smid_col)
    p2 = _edge_kernel(ids_p, x2s, zeros_nd)
    out_pad = _combine(p2, sfin_col)
    return out_pad[:N]


# speedup vs baseline: 12.6439x; 1.0035x over previous
import functools

import jax
import jax.numpy as jnp
from jax import lax
from jax.experimental import pallas as pl
from jax.experimental.pallas import tpu as pltpu
from jax.experimental.pallas import tpu_sc as plsc

N = 10000
E = 320000
D = 128

NC = 2
NS = 16
NW = NC * NS

CHUNK = 80
N_PAD = 10240
RPT = N_PAD // NS
EPW = 10240
NCHUNK = EPW // CHUNK
IG = 8
NG = NCHUNK // IG
NVEC = EPW // 16
E_PAD = NW * EPW

_MESH = plsc.VectorSubcoreMesh(core_axis_name="c", subcore_axis_name="s")


@functools.partial(
    pl.kernel,
    out_type=jax.ShapeDtypeStruct((NW, 2, N_PAD), jnp.float32),
    mesh=_MESH,
    compiler_params=pltpu.CompilerParams(needs_layout_passes=False),
    scratch_types=[
        pltpu.VMEM((EPW,), jnp.int32),
        pltpu.VMEM((EPW,), jnp.int32),
        pltpu.VMEM((N_PAD,), jnp.float32),
        pltpu.VMEM((N_PAD,), jnp.float32),
    ],
)
def _deg_kernel(src_hbm, dst_hbm, zeros_hbm, hp_hbm,
                src_v, dst_v, hist_s, hist_d):
    cid = lax.axis_index("c")
    sid = lax.axis_index("s")
    wid = cid * NS + sid

    pltpu.sync_copy(src_hbm.at[wid], src_v)
    pltpu.sync_copy(dst_hbm.at[wid], dst_v)
    pltpu.sync_copy(zeros_hbm, hist_s)
    pltpu.sync_copy(zeros_hbm, hist_d)
    ones = jnp.full((16,), 1.0, jnp.float32)

    def body(j, carry):
        plsc.addupdate_scatter(hist_s, [src_v[pl.ds(j * 16, 16)]], ones)
        plsc.addupdate_scatter(hist_d, [dst_v[pl.ds(j * 16, 16)]], ones)
        return carry

    lax.fori_loop(0, NVEC, body, 0)
    pltpu.sync_copy(hist_s, hp_hbm.at[wid, 0])
    pltpu.sync_copy(hist_d, hp_hbm.at[wid, 1])


NBUF = 4

@functools.partial(
    pl.kernel,
    out_type=jax.ShapeDtypeStruct((NC, N_PAD, D), jnp.float32),
    mesh=_MESH,
    scratch_types=[
        pltpu.VMEM((2, IG, CHUNK), jnp.int32),
        pltpu.VMEM((2, IG, CHUNK), jnp.int32),
        pltpu.VMEM((CHUNK, D), jnp.float32),
        pltpu.VMEM((CHUNK, D), jnp.float32),
        pltpu.VMEM((CHUNK, D), jnp.float32),
        pltpu.VMEM((CHUNK, D), jnp.float32),
        pltpu.VMEM_SHARED((N_PAD, D), jnp.float32),
        pltpu.SemaphoreType.DMA,
        pltpu.SemaphoreType.DMA,
        pltpu.SemaphoreType.DMA,
        pltpu.SemaphoreType.DMA,
        pltpu.SemaphoreType.DMA,
        pltpu.SemaphoreType.DMA,
    ],
)
def _edge_kernel(ids_hbm, x_hbm, zeros_hbm, p_hbm,
                 idb_0, idb_1, rows_0, rows_1, rows_2, rows_3, acc,
                 gsem_0, gsem_1, gsem_2, gsem_3, isem_0, isem_1):
    idb = (idb_0, idb_1)
    rows_v = (rows_0, rows_1, rows_2, rows_3)
    gsem = (gsem_0, gsem_1, gsem_2, gsem_3)
    isem = (isem_0, isem_1)
    cid = lax.axis_index("c")
    sid = lax.axis_index("s")
    wid = cid * NS + sid

    rows = pl.ds(sid * RPT, RPT)
    pltpu.sync_copy(zeros_hbm.at[rows], acc.at[rows])
    pltpu.sync_copy(ids_hbm.at[wid, 0], idb_0)
    pltpu.async_copy(ids_hbm.at[wid, 1], idb_1, isem_1)
    plsc.subcore_barrier()

    for b in range(NBUF):
        pltpu.async_copy(x_hbm.at[idb_0.at[0, b]], rows_v[b], gsem[b])

    def body(p, carry):
        for gg in range(2):
            idc = idb[gg]
            idn = idb[1 - gg]
            pltpu.make_async_copy(
                ids_hbm.at[wid, 0], idn, isem[1 - gg]).wait()
            for k in range(IG):
                b = k % NBUF
                pltpu.make_async_copy(
                    x_hbm.at[idc.at[0, k]], rows_v[b], gsem[b]).wait()
                pltpu.sync_copy(rows_v[b], acc.at[idc.at[1, k]], add=True)
                if k < IG - NBUF:
                    pltpu.async_copy(
                        x_hbm.at[idc.at[0, k + NBUF]], rows_v[b], gsem[b])
                else:
                    pltpu.async_copy(
                        x_hbm.at[idn.at[0, k + NBUF - IG]], rows_v[b],
                        gsem[b])
            gnext = jnp.minimum(2 * p + 2 + gg, NG - 1)
            pltpu.async_copy(ids_hbm.at[wid, gnext], idc, isem[gg])
        return carry

    lax.fori_loop(0, NG // 2, body, 0)

    pltpu.make_async_copy(ids_hbm.at[wid, 0], idb_1, isem_1).wait()
    for b in range(NBUF):
        pltpu.make_async_copy(
            x_hbm.at[idb_0.at[0, b]], rows_v[b], gsem[b]).wait()

    plsc.subcore_barrier()
    pltpu.sync_copy(acc.at[rows], p_hbm.at[cid, rows])


def _norm_prescale_body(hp_ref, x_ref, x1s_ref, smid_ref, sfin_ref):
    h = jnp.sum(hp_ref[...], axis=0)
    no = lax.rsqrt(jnp.clip(h[0], 1.0, None))
    ni = lax.rsqrt(jnp.clip(h[1], 1.0, None))
    x1s_ref[...] = x_ref[...] * no[:, None]
    smid_ref[...] = (no * ni)[:, None]
    sfin_ref[...] = ni[:, None]


def _norm_prescale(hp, x_pad):
    col = jax.ShapeDtypeStruct((N_PAD, 1), jnp.float32)
    return pl.pallas_call(
        _norm_prescale_body,
        grid=(NS,),
        in_specs=[
            pl.BlockSpec((NW, 2, RPT), lambda i: (0, 0, i)),
            pl.BlockSpec((RPT, D), lambda i: (i, 0)),
        ],
        out_specs=[
            pl.BlockSpec((RPT, D), lambda i: (i, 0)),
            pl.BlockSpec((RPT, 1), lambda i: (i, 0)),
            pl.BlockSpec((RPT, 1), lambda i: (i, 0)),
        ],
        out_shape=[jax.ShapeDtypeStruct((N_PAD, D), jnp.float32), col, col],
    )(hp, x_pad)


def _combine_body(p_ref, s_ref, out_ref):
    pb = p_ref[...]
    out_ref[...] = (pb[0] + pb[1]) * s_ref[...]


def _combine(p, s_col):
    return pl.pallas_call(
        _combine_body,
        grid=(NS,),
        in_specs=[
            pl.BlockSpec((NC, RPT, D), lambda i: (0, i, 0)),
            pl.BlockSpec((RPT, 1), lambda i: (i, 0)),
        ],
        out_specs=pl.BlockSpec((RPT, D), lambda i: (i, 0)),
        out_shape=jax.ShapeDtypeStruct((N_PAD, D), jnp.float32),
    )(p, s_col)


def kernel(adj_t, x):
    src = adj_t[0].astype(jnp.int32)
    dst = adj_t[1].astype(jnp.int32)
    pad_ids = N + (jnp.arange(E_PAD - E, dtype=jnp.int32) % 128)
    src_flat = jnp.concatenate([src, pad_ids]).reshape(NW, EPW)
    dst_flat = jnp.concatenate([dst, pad_ids]).reshape(NW, EPW)
    ids_p = jnp.stack([src_flat.reshape(NW, NG, IG, CHUNK),
                       dst_flat.reshape(NW, NG, IG, CHUNK)], axis=2)
    x_pad = jnp.concatenate(
        [x.astype(jnp.float32), jnp.zeros((N_PAD - N, D), jnp.float32)])

    zeros_n = jnp.zeros((N_PAD,), jnp.float32)
    zeros_nd = jnp.zeros((N_PAD, D), jnp.float32)

    hp = _deg_kernel(src_flat, dst_flat, zeros_n)
    x1s, smid_col, sfin_col = _norm_prescale(hp, x_pad)
    p1 = _edge_kernel(ids_p, x1s, zeros_nd)
    x2s = _combine(p1, smid_col)
    p2 = _edge_kernel(ids_p, x2s, zeros_nd)
    out_pad = _combine(p2, sfin_col)
    return out_pad[:N]
